# baseline (speedup 1.0000x reference)
import functools

import jax
import jax.numpy as jnp
from jax import lax
from jax.experimental import pallas as pl
from jax.experimental.pallas import tpu as pltpu
from jax.experimental.pallas import tpu_sc as plsc

MAX_MASKS = 5
RING_K = 7
RING_R = RING_K // 2
SC_LANES = 16


def _topk(mask_scores, m):
    b_dim, k_dim = mask_scores.shape
    pad = (-k_dim) % SC_LANES
    kp = k_dim + pad
    nv = kp // SC_LANES
    scores_p = jnp.pad(mask_scores.astype(jnp.float32), ((0, 0), (0, pad)),
                       constant_values=-jnp.inf)

    mesh = plsc.VectorSubcoreMesh(core_axis_name="c", subcore_axis_name="s")

    @functools.partial(
        pl.kernel, mesh=mesh,
        out_type=jax.ShapeDtypeStruct((b_dim, SC_LANES), jnp.int32),
        scratch_types=[
            pltpu.VMEM((kp,), jnp.float32),
            pltpu.VMEM((SC_LANES,), jnp.int32),
        ],
    )
    def topk_sc(scores_hbm, out_hbm, row_v, idx_v):
        nc = plsc.get_sparse_core_info().num_cores
        wid = lax.axis_index("s") * nc + lax.axis_index("c")
        lane = lax.iota(jnp.int32, SC_LANES)

        def allreduce(x, op):
            for s in (8, 4, 2, 1):
                perm = jnp.bitwise_xor(lane, s)
                x = op(x, x.at[perm].get(mode="promise_in_bounds"))
            return x

        for b in range(b_dim):
            @pl.when(wid == b)
            def _(b=b):
                pltpu.sync_copy(scores_hbm.at[b], row_v)
                vals = [row_v[pl.ds(SC_LANES * j, SC_LANES)]
                        for j in range(nv)]
                taken = [jnp.zeros((SC_LANES,), jnp.int32) for _ in range(nv)]
                idxvec = jnp.zeros((SC_LANES,), jnp.int32)
                for t in range(m):
                    masked = [jnp.where(taken[j] > 0, -jnp.inf, vals[j])
                              for j in range(nv)]
                    mx = masked[0]
                    for j in range(1, nv):
                        mx = jnp.maximum(mx, masked[j])
                    mxs = allreduce(mx, jnp.maximum)
                    cand = [jnp.where((masked[j] == mxs) & (taken[j] == 0),
                                      lane + SC_LANES * j, kp)
                            for j in range(nv)]
                    cmin = cand[0]
                    for j in range(1, nv):
                        cmin = jnp.minimum(cmin, cand[j])
                    amx = allreduce(cmin, jnp.minimum)
                    taken = [jnp.where(lane + SC_LANES * j == amx, 1, taken[j])
                             for j in range(nv)]
                    idxvec = jnp.where(lane == t, amx, idxvec)
                idx_v[...] = idxvec
                pltpu.sync_copy(idx_v, out_hbm.at[b])

    return topk_sc(scores_p)[:, :m]


def _dilate_1d(p, axis, radius):
    neg = jnp.full_like(p, -jnp.inf)

    def shifted_max(x, d):
        if axis == 0:
            up = jnp.concatenate([x[d:, :], neg[:d, :]], axis=0)
            dn = jnp.concatenate([neg[:d, :], x[:-d, :]], axis=0)
        else:
            up = jnp.concatenate([x[:, d:], neg[:, :d]], axis=1)
            dn = jnp.concatenate([neg[:, :d], x[:, :-d]], axis=1)
        return jnp.maximum(x, jnp.maximum(up, dn))

    assert radius == 3
    return shifted_max(shifted_max(p, 1), 2)


def _fused_body(idx_ref, *refs, m, nchunk, hchunk):
    ml_refs = refs[:m]
    pf_ref, wev_ref, bev_ref, out_ref, wpl_ref, acc_ref, tpf_ref = refs[m:]
    c = pl.program_id(1)

    @pl.when(c == 0)
    def _():
        acc_ref[...] = jnp.zeros_like(acc_ref)
        for j, mlr in enumerate(ml_refs):
            x = mlr[0, 0]
            h, w = x.shape
            p = jax.nn.sigmoid(x)
            dil = _dilate_1d(_dilate_1d(p, 0, RING_R), 1, RING_R)
            ring = jnp.maximum(dil - p, 0.0)
            sp = jnp.sum(p)
            wpl_ref[j] = p / jnp.maximum(sp, 1e-6)
            sr = jnp.sum(ring)
            empty = (sr == 0.0).astype(jnp.float32)
            ring = ring + empty * 1e-4
            sr2 = sr + empty * (1e-4 * h * w)
            wpl_ref[m + j] = ring / jnp.maximum(sr2, 1e-6)

    base = c * hchunk
    tpf_ref[...] = jnp.transpose(pf_ref[0], (1, 0, 2))
    s = jnp.zeros_like(acc_ref)
    for h in range(hchunk):
        w2 = wpl_ref[:, base + h, :]
        s = s + jax.lax.dot_general(
            w2, tpf_ref[h], (((1,), (1,)), ((), ())),
            preferred_element_type=jnp.float32)
    acc_ref[...] += s

    @pl.when(c == nchunk - 1)
    def _():
        ev = jax.lax.dot_general(
            acc_ref[...], wev_ref[...], (((1,), (1,)), ((), ())),
            preferred_element_type=jnp.float32)
        out_ref[0] = ev + bev_ref[...]


def _fused(mask_logits, idx, pixel_feat, w_ev, b_ev2, m, hchunk):
    b_dim, k_dim, h, w = mask_logits.shape
    c_dim = pixel_feat.shape[1]
    d_dim = w_ev.shape[0]
    nchunk = h // hchunk

    def mask_map(j):
        return lambda b, c, idx_ref: (b, idx_ref[b, j], 0, 0)

    in_specs = [pl.BlockSpec((1, 1, h, w), mask_map(j)) for j in range(m)]
    in_specs += [
        pl.BlockSpec((1, c_dim, hchunk, w), lambda b, c, idx_ref: (b, 0, c, 0)),
        pl.BlockSpec((d_dim, c_dim), lambda b, c, idx_ref: (0, 0)),
        pl.BlockSpec((1, d_dim), lambda b, c, idx_ref: (0, 0)),
    ]
    grid_spec = pltpu.PrefetchScalarGridSpec(
        num_scalar_prefetch=1,
        grid=(b_dim, nchunk),
        in_specs=in_specs,
        out_specs=[pl.BlockSpec((1, 2 * m, d_dim),
                                lambda b, c, idx_ref: (b, 0, 0))],
        scratch_shapes=[
            pltpu.VMEM((2 * m, h, w), jnp.float32),
            pltpu.VMEM((2 * m, c_dim), jnp.float32),
            pltpu.VMEM((hchunk, c_dim, w), jnp.float32),
        ],
    )
    return pl.pallas_call(
        functools.partial(_fused_body, m=m, nchunk=nchunk, hchunk=hchunk),
        grid_spec=grid_spec,
        out_shape=[jax.ShapeDtypeStruct((b_dim, 2 * m, d_dim), jnp.float32)],
        compiler_params=pltpu.CompilerParams(
            dimension_semantics=("parallel", "arbitrary")),
    )(idx, *([mask_logits] * m), pixel_feat, w_ev, b_ev2)[0]


def kernel(mask_logits, pixel_feat, mask_scores, W_ev, b_ev):
    b_dim, k_dim, h, w = mask_logits.shape
    m = min(MAX_MASKS, k_dim)

    idx = _topk(mask_scores, m)
    ev = _fused(mask_logits, idx, pixel_feat, W_ev, b_ev.reshape(1, -1),
                m, hchunk=56)
    return ev
